# baseline (speedup 1.0000x reference)
import functools

import jax
import jax.numpy as jnp
from jax import lax
from jax.experimental import pallas as pl
from jax.experimental.pallas import tpu as pltpu
from jax.experimental.pallas import tpu_sc as plsc

NUM_NODES = 1000000
EMBED_DIM = 64

_NBATCH = 16384
_S = 50
_SP = 56
_NW = 32
_BPW = _NBATCH // _NW
_SB = 8
_NB = 2
_T = _BPW // (_SB * _NB)


def _make_gather():
    mesh = plsc.VectorSubcoreMesh(core_axis_name="c", subcore_axis_name="s")

    @functools.partial(
        pl.kernel,
        out_type=jax.ShapeDtypeStruct((_NBATCH, _S, EMBED_DIM), jnp.float32),
        mesh=mesh,
        scratch_types=(
            [pltpu.VMEM((_BPW * _SP,), jnp.int32)]
            + [pltpu.VMEM((_SB, _S, EMBED_DIM), jnp.float32)] * _NB
            + [pltpu.SemaphoreType.DMA] * (2 * _NB)
        ),
        compiler_params=pltpu.CompilerParams(use_tc_tiling_on_sc=False),
    )
    def gather_kernel(idx_hbm, table_hbm, out_hbm, idx_v, *bufs_and_sems):
        bufs = bufs_and_sems[:_NB]
        gsem = bufs_and_sems[_NB:2 * _NB]
        ssem = bufs_and_sems[2 * _NB:]

        wid = lax.axis_index("s") * 2 + lax.axis_index("c")
        b_base = wid * _BPW
        pltpu.sync_copy(idx_hbm.at[pl.ds(b_base * _SP, _BPW * _SP)], idx_v)

        def issue_gathers(blk, b):
            for k in range(_SB):
                pltpu.async_copy(
                    table_hbm.at[idx_v.at[pl.ds((blk * _SB + k) * _SP, _S)]],
                    bufs[b].at[k],
                    gsem[b])

        for b in range(_NB):
            issue_gathers(b, b)

        def body(t, carry):
            for b in range(_NB):
                blk = t * _NB + b
                for k in range(_SB):
                    pltpu.make_async_copy(
                        table_hbm.at[idx_v.at[pl.ds(0, _S)]],
                        bufs[b].at[k],
                        gsem[b]).wait()
                pltpu.async_copy(
                    bufs[b],
                    out_hbm.at[pl.ds(b_base + blk * _SB, _SB)],
                    ssem[b])
            for b in range(_NB):
                pltpu.make_async_copy(
                    bufs[b], out_hbm.at[pl.ds(0, _SB)], ssem[b]).wait()
                pl.when(t != _T - 1)(
                    functools.partial(issue_gathers, (t + 1) * _NB + b, b))
            return carry

        lax.fori_loop(0, _T, body, 0, unroll=False)

    return gather_kernel


_gather = _make_gather()


def kernel(indices, weight):
    idxp = jnp.pad(indices.astype(jnp.int32), ((0, 0), (0, _SP - _S)))
    return _gather(idxp.reshape(-1), weight)


# speedup vs baseline: 1.0046x; 1.0046x over previous
import functools

import jax
import jax.numpy as jnp
from jax import lax
from jax.experimental import pallas as pl
from jax.experimental.pallas import tpu as pltpu
from jax.experimental.pallas import tpu_sc as plsc

NUM_NODES = 1000000
EMBED_DIM = 64

_NBATCH = 16384
_S = 50
_SP = 56
_NW = 32
_BPW = _NBATCH // _NW
_SB = 4
_NB = 4
_T = _BPW // (_SB * _NB)


def _make_gather():
    mesh = plsc.VectorSubcoreMesh(core_axis_name="c", subcore_axis_name="s")

    @functools.partial(
        pl.kernel,
        out_type=jax.ShapeDtypeStruct((_NBATCH, _S, EMBED_DIM), jnp.float32),
        mesh=mesh,
        scratch_types=(
            [pltpu.VMEM((_BPW * _SP,), jnp.int32)]
            + [pltpu.VMEM((_SB, _S, EMBED_DIM), jnp.float32)] * _NB
            + [pltpu.SemaphoreType.DMA] * (2 * _NB)
        ),
        compiler_params=pltpu.CompilerParams(use_tc_tiling_on_sc=False),
    )
    def gather_kernel(idx_hbm, table_hbm, out_hbm, idx_v, *bufs_and_sems):
        bufs = bufs_and_sems[:_NB]
        gsem = bufs_and_sems[_NB:2 * _NB]
        ssem = bufs_and_sems[2 * _NB:]

        wid = lax.axis_index("s") * 2 + lax.axis_index("c")
        b_base = wid * _BPW
        pltpu.sync_copy(idx_hbm.at[pl.ds(b_base * _SP, _BPW * _SP)], idx_v)

        def issue_gathers(blk, b):
            for k in range(_SB):
                pltpu.async_copy(
                    table_hbm.at[idx_v.at[pl.ds((blk * _SB + k) * _SP, _S)]],
                    bufs[b].at[k],
                    gsem[b])

        for b in range(_NB):
            issue_gathers(b, b)

        def body(t, carry):
            for b in range(_NB):
                blk = t * _NB + b
                for k in range(_SB):
                    pltpu.make_async_copy(
                        table_hbm.at[idx_v.at[pl.ds(0, _S)]],
                        bufs[b].at[k],
                        gsem[b]).wait()
                pltpu.async_copy(
                    bufs[b],
                    out_hbm.at[pl.ds(b_base + blk * _SB, _SB)],
                    ssem[b])
            for b in range(_NB):
                pltpu.make_async_copy(
                    bufs[b], out_hbm.at[pl.ds(0, _SB)], ssem[b]).wait()
                pl.when(t != _T - 1)(
                    functools.partial(issue_gathers, (t + 1) * _NB + b, b))
            return carry

        lax.fori_loop(0, _T, body, 0, unroll=False)

    return gather_kernel


_gather = _make_gather()


def kernel(indices, weight):
    idxp = jnp.pad(indices.astype(jnp.int32), ((0, 0), (0, _SP - _S)))
    return _gather(idxp.reshape(-1), weight)


# speedup vs baseline: 1.0068x; 1.0022x over previous
import functools

import jax
import jax.numpy as jnp
from jax import lax
from jax.experimental import pallas as pl
from jax.experimental.pallas import tpu as pltpu
from jax.experimental.pallas import tpu_sc as plsc

NUM_NODES = 1000000
EMBED_DIM = 64

_NBATCH = 16384
_S = 50
_SP = 56
_NW = 32
_BPW = _NBATCH // _NW
_SB = 4
_NB = 4
_T = _BPW // (_SB * _NB)


def _make_gather():
    mesh = plsc.VectorSubcoreMesh(core_axis_name="c", subcore_axis_name="s")

    @functools.partial(
        pl.kernel,
        out_type=jax.ShapeDtypeStruct((_NBATCH, _S, EMBED_DIM), jnp.float32),
        mesh=mesh,
        scratch_types=(
            [pltpu.VMEM((_BPW * _SP,), jnp.int32)]
            + [pltpu.VMEM((_SB, _S, EMBED_DIM), jnp.float32)] * _NB
            + [pltpu.SemaphoreType.DMA] * (2 * _NB)
        ),
        compiler_params=pltpu.CompilerParams(use_tc_tiling_on_sc=False),
    )
    def gather_kernel(idx_hbm, table_hbm, out_hbm, idx_v, *bufs_and_sems):
        bufs = bufs_and_sems[:_NB]
        gsem = bufs_and_sems[_NB:2 * _NB]
        ssem = bufs_and_sems[2 * _NB:]

        wid = lax.axis_index("s") * 2 + lax.axis_index("c")
        b_base = wid * _BPW
        pltpu.sync_copy(idx_hbm.at[pl.ds(b_base * _SP, _BPW * _SP)], idx_v)

        def issue_gathers(blk, b):
            for k in range(_SB):
                pltpu.async_copy(
                    table_hbm.at[idx_v.at[pl.ds((blk * _SB + k) * _SP, _S)]],
                    bufs[b].at[k],
                    gsem[b])

        for b in range(_NB):
            issue_gathers(b, b)

        def body(t, carry):
            for b in range(_NB):
                blk = t * _NB + b
                for k in range(_SB):
                    pltpu.make_async_copy(
                        table_hbm.at[idx_v.at[pl.ds(0, _S)]],
                        bufs[b].at[k],
                        gsem[b]).wait()
                pltpu.async_copy(
                    bufs[b],
                    out_hbm.at[pl.ds(b_base + blk * _SB, _SB)],
                    ssem[b])
            for b in range(_NB):
                pltpu.make_async_copy(
                    bufs[b], out_hbm.at[pl.ds(0, _SB)], ssem[b]).wait()
                pl.when(t != _T - 1)(
                    functools.partial(issue_gathers, (t + 1) * _NB + b, b))
            return carry

        lax.fori_loop(0, _T, body, 0, unroll=False)

    return gather_kernel


_gather_cache = []


def kernel(indices, weight):
    if not _gather_cache:
        _gather_cache.append(_make_gather())
    idxp = jnp.pad(indices.astype(jnp.int32), ((0, 0), (0, _SP - _S)))
    return _gather_cache[0](idxp.reshape(-1), weight)
